# baseline (speedup 1.0000x reference)
import functools

import jax
import jax.numpy as jnp
from jax import lax
from jax.experimental import pallas as pl
from jax.experimental.pallas import tpu as pltpu
from jax.experimental.pallas import tpu_sc as plsc

_N_NODES = 50000
_N_EDGES = 1600000
_N_UNARY = 8
_N_BINARY = 2

_NC = 2
_NS = 16
_NW = _NC * _NS

_B_PER_W = _N_EDGES // _NW
_CHUNK = 200
_NCHUNK = _B_PER_W // _CHUNK
_NBUF = 4
_NGROUP = _NCHUNK // _NBUF
_NTAIL = _NCHUNK - _NGROUP * _NBUF

_U_GRID = 25
_U_BLOCK = _N_NODES // _U_GRID


def _sc_copy_binary(b_hbm, bo_hbm, b0, b1, b2, b3, sem_in, sem_out):
    wid = lax.axis_index("s") * _NC + lax.axis_index("c")
    base = wid * _B_PER_W
    bufs = (b0, b1, b2, b3)

    def in_copy(c, buf):
        return pltpu.make_async_copy(
            b_hbm.at[pl.ds(base + c * _CHUNK, _CHUNK)], buf, sem_in
        )

    def out_copy(c, buf):
        return pltpu.make_async_copy(
            buf, bo_hbm.at[pl.ds(base + c * _CHUNK, _CHUNK)], sem_out
        )

    for b in range(_NBUF):
        in_copy(b, bufs[b]).start()

    def group(g, carry):
        for b in range(_NBUF):
            c = g * _NBUF + b
            in_copy(c, bufs[b]).wait()
            out_copy(c, bufs[b]).start()

        @pl.when(g < _NGROUP - 1)
        def _():
            for b in range(_NBUF):
                c = g * _NBUF + b
                out_copy(c, bufs[b]).wait()
                in_copy(c + _NBUF, bufs[b]).start()

        @pl.when(g == _NGROUP - 1)
        def _():
            for b in range(_NBUF):
                out_copy(g * _NBUF + b, bufs[b]).wait()

        return carry

    lax.fori_loop(0, _NGROUP, group, 0)

    for t in range(_NTAIL):
        c = _NGROUP * _NBUF + t
        in_copy(c, bufs[t]).start()
    for t in range(_NTAIL):
        c = _NGROUP * _NBUF + t
        in_copy(c, bufs[t]).wait()
        out_copy(c, bufs[t]).start()
    for t in range(_NTAIL):
        out_copy(_NGROUP * _NBUF + t, bufs[t]).wait()


def _tc_copy_unary(u_ref, uo_ref):
    uo_ref[...] = u_ref[...]


def kernel(unary, binary, index1, index2):
    mesh = plsc.VectorSubcoreMesh(core_axis_name="c", subcore_axis_name="s")
    run = functools.partial(
        pl.kernel,
        mesh=mesh,
        out_type=jax.ShapeDtypeStruct(binary.shape, binary.dtype),
        scratch_types=[
            pltpu.VMEM((_CHUNK, _N_BINARY), jnp.float32),
            pltpu.VMEM((_CHUNK, _N_BINARY), jnp.float32),
            pltpu.VMEM((_CHUNK, _N_BINARY), jnp.float32),
            pltpu.VMEM((_CHUNK, _N_BINARY), jnp.float32),
            pltpu.SemaphoreType.DMA,
            pltpu.SemaphoreType.DMA,
        ],
        compiler_params=pltpu.CompilerParams(use_tc_tiling_on_sc=True),
    )(_sc_copy_binary)
    bo = run(binary)

    uo = pl.pallas_call(
        _tc_copy_unary,
        grid=(_U_GRID,),
        in_specs=[pl.BlockSpec((_U_BLOCK, _N_UNARY), lambda i: (i, 0))],
        out_specs=pl.BlockSpec((_U_BLOCK, _N_UNARY), lambda i: (i, 0)),
        out_shape=jax.ShapeDtypeStruct(unary.shape, unary.dtype),
        compiler_params=pltpu.CompilerParams(
            dimension_semantics=("arbitrary",),
        ),
    )(unary)
    return (uo, bo)
